# baseline (speedup 1.0000x reference)
import jax
import jax.numpy as jnp
from jax.experimental import pallas as pl
from jax.experimental.pallas import tpu as pltpu

_VQ = 8
_K = 1024
_D = 256
_N = 8192
_H = 256
_NH = 1
_BT = _H * _NH


def _vq_body(gate_ref, x_ref, cb_ref, quant_ref, idx_ref, sse_ref, km_ref,
             e2_ref, ehi_ref, emid_ref, elo_ref):
    @pl.when(pl.program_id(0) == 0)
    def _init_scratch():
        for i in range(_VQ):
            e = cb_ref[i]
            e2_ref[i, :] = jnp.sum(e * e, axis=1)
            hi = e.astype(jnp.bfloat16)
            rm = e - hi.astype(jnp.float32)
            mid = rm.astype(jnp.bfloat16)
            lo = (rm - mid.astype(jnp.float32)).astype(jnp.bfloat16)
            ehi_ref[i] = hi
            emid_ref[i] = mid
            elo_ref[i] = lo

    gate = gate_ref[0]
    lane_iota = jax.lax.broadcasted_iota(jnp.int32, (_H, _K), 1)
    sse = jnp.zeros((1, 1), jnp.float32)
    r = [x_ref[pl.ds(h * _H, _H), :] * gate for h in range(_NH)]
    quant = [None] * _NH
    for i in range(_VQ):
        e = cb_ref[i]
        e2 = e2_ref[i, :]
        for h in range(_NH):
            rh = r[h]
            km_ref[i, pl.ds(h * _H, _H), :] = rh
            r2 = jnp.sum(rh * rh, axis=1, keepdims=True)
            d = (r2 - jnp.dot(rh + rh, e.T, preferred_element_type=jnp.float32)
                 + e2[None, :])
            dmin = jnp.min(d, axis=1, keepdims=True)
            idx = jnp.min(jnp.where(d == dmin, lane_iota, _K), axis=1)
            idx_ref[i, pl.ds(h * _H, _H)] = idx.astype(jnp.int32)
            oh = (lane_iota == idx[:, None]).astype(jnp.float32).astype(jnp.bfloat16)
            q = ((jnp.dot(oh, ehi_ref[i], preferred_element_type=jnp.float32)
                  + jnp.dot(oh, emid_ref[i], preferred_element_type=jnp.float32))
                 + jnp.dot(oh, elo_ref[i], preferred_element_type=jnp.float32))
            diff = q - rh
            sse += jnp.sum(diff * diff).reshape(1, 1)
            q_st = rh + diff
            quant[h] = q_st if i == 0 else quant[h] + q_st
            r[h] = rh - q_st
    for h in range(_NH):
        quant_ref[pl.ds(h * _H, _H), :] = quant[h]

    @pl.when(pl.program_id(0) == 0)
    def _init():
        sse_ref[...] = sse

    @pl.when(pl.program_id(0) != 0)
    def _acc():
        sse_ref[...] += sse


def kernel(x, skip_vq, codebooks):
    gate = (1 - jnp.asarray(skip_vq)).astype(x.dtype).reshape(1)
    xf = x.reshape(_N, _D)
    grid = (_N // _BT,)
    quant, idx, sse, km = pl.pallas_call(
        _vq_body,
        grid=grid,
        in_specs=[
            pl.BlockSpec(memory_space=pltpu.SMEM),
            pl.BlockSpec((_BT, _D), lambda t: (t, 0)),
            pl.BlockSpec((_VQ, _K, _D), lambda t: (0, 0, 0)),
        ],
        out_specs=[
            pl.BlockSpec((_BT, _D), lambda t: (t, 0)),
            pl.BlockSpec((_VQ, _BT), lambda t: (0, t)),
            pl.BlockSpec((1, 1), lambda t: (0, 0)),
            pl.BlockSpec((_VQ, _BT, _D), lambda t: (0, t, 0)),
        ],
        out_shape=[
            jax.ShapeDtypeStruct((_N, _D), jnp.float32),
            jax.ShapeDtypeStruct((_VQ, _N), jnp.int32),
            jax.ShapeDtypeStruct((1, 1), jnp.float32),
            jax.ShapeDtypeStruct((_VQ, _N, _D), jnp.float32),
        ],
        scratch_shapes=[pltpu.VMEM((_VQ, _K), jnp.float32),
                        pltpu.VMEM((_VQ, _K, _D), jnp.bfloat16),
                        pltpu.VMEM((_VQ, _K, _D), jnp.bfloat16),
                        pltpu.VMEM((_VQ, _K, _D), jnp.bfloat16)],
    )(gate, xf, codebooks)
    quantized = quant.reshape(x.shape)
    indices = idx.reshape(_VQ, x.shape[0], x.shape[1])
    vq_loss = (sse / (_N * _D)).reshape(())
    kmeans_inputs = km.reshape(_VQ, x.shape[0], x.shape[1], _D)
    return (quantized, indices, vq_loss, kmeans_inputs)


# speedup vs baseline: 1.6236x; 1.6236x over previous
import jax
import jax.numpy as jnp
from jax.experimental import pallas as pl
from jax.experimental.pallas import tpu as pltpu

_VQ = 8
_K = 1024
_D = 256
_N = 8192
_H = 256
_NH = 4
_BT = _H * _NH


def _vq_body(gate_ref, x_ref, cb_ref, quant_ref, idx_ref, sse_ref, km_ref,
             e2_ref, ehi_ref, emid_ref, elo_ref):
    @pl.when(pl.program_id(0) == 0)
    def _init_scratch():
        for i in range(_VQ):
            e = cb_ref[i]
            e2_ref[i, :] = jnp.sum(e * e, axis=1)
            hi = e.astype(jnp.bfloat16)
            rm = e - hi.astype(jnp.float32)
            mid = rm.astype(jnp.bfloat16)
            lo = (rm - mid.astype(jnp.float32)).astype(jnp.bfloat16)
            ehi_ref[i] = hi
            emid_ref[i] = mid
            elo_ref[i] = lo

    gate = gate_ref[0]
    lane_iota = jax.lax.broadcasted_iota(jnp.int32, (_H, _K), 1)
    sse = jnp.zeros((1, 1), jnp.float32)
    r = [x_ref[pl.ds(h * _H, _H), :] * gate for h in range(_NH)]
    quant = [None] * _NH
    for i in range(_VQ):
        e = cb_ref[i]
        e2 = e2_ref[i, :]
        for h in range(_NH):
            rh = r[h]
            km_ref[i, pl.ds(h * _H, _H), :] = rh
            r2 = jnp.sum(rh * rh, axis=1, keepdims=True)
            d = (r2 - jnp.dot(rh + rh, e.T, preferred_element_type=jnp.float32)
                 + e2[None, :])
            dmin = jnp.min(d, axis=1, keepdims=True)
            idx = jnp.min(jnp.where(d == dmin, lane_iota, _K), axis=1)
            idx_ref[i, pl.ds(h * _H, _H)] = idx.astype(jnp.int32)
            oh = (lane_iota == idx[:, None]).astype(jnp.float32).astype(jnp.bfloat16)
            q = ((jnp.dot(oh, ehi_ref[i], preferred_element_type=jnp.float32)
                  + jnp.dot(oh, emid_ref[i], preferred_element_type=jnp.float32))
                 + jnp.dot(oh, elo_ref[i], preferred_element_type=jnp.float32))
            diff = q - rh
            sse += jnp.sum(diff * diff).reshape(1, 1)
            q_st = rh + diff
            quant[h] = q_st if i == 0 else quant[h] + q_st
            r[h] = rh - q_st
    for h in range(_NH):
        quant_ref[pl.ds(h * _H, _H), :] = quant[h]

    @pl.when(pl.program_id(0) == 0)
    def _init():
        sse_ref[...] = sse

    @pl.when(pl.program_id(0) != 0)
    def _acc():
        sse_ref[...] += sse


def kernel(x, skip_vq, codebooks):
    gate = (1 - jnp.asarray(skip_vq)).astype(x.dtype).reshape(1)
    xf = x.reshape(_N, _D)
    grid = (_N // _BT,)
    quant, idx, sse, km = pl.pallas_call(
        _vq_body,
        grid=grid,
        in_specs=[
            pl.BlockSpec(memory_space=pltpu.SMEM),
            pl.BlockSpec((_BT, _D), lambda t: (t, 0)),
            pl.BlockSpec((_VQ, _K, _D), lambda t: (0, 0, 0)),
        ],
        out_specs=[
            pl.BlockSpec((_BT, _D), lambda t: (t, 0)),
            pl.BlockSpec((_VQ, _BT), lambda t: (0, t)),
            pl.BlockSpec((1, 1), lambda t: (0, 0)),
            pl.BlockSpec((_VQ, _BT, _D), lambda t: (0, t, 0)),
        ],
        out_shape=[
            jax.ShapeDtypeStruct((_N, _D), jnp.float32),
            jax.ShapeDtypeStruct((_VQ, _N), jnp.int32),
            jax.ShapeDtypeStruct((1, 1), jnp.float32),
            jax.ShapeDtypeStruct((_VQ, _N, _D), jnp.float32),
        ],
        scratch_shapes=[pltpu.VMEM((_VQ, _K), jnp.float32),
                        pltpu.VMEM((_VQ, _K, _D), jnp.bfloat16),
                        pltpu.VMEM((_VQ, _K, _D), jnp.bfloat16),
                        pltpu.VMEM((_VQ, _K, _D), jnp.bfloat16)],
    )(gate, xf, codebooks)
    quantized = quant.reshape(x.shape)
    indices = idx.reshape(_VQ, x.shape[0], x.shape[1])
    vq_loss = (sse / (_N * _D)).reshape(())
    kmeans_inputs = km.reshape(_VQ, x.shape[0], x.shape[1], _D)
    return (quantized, indices, vq_loss, kmeans_inputs)


# speedup vs baseline: 1.7388x; 1.0709x over previous
import jax
import jax.numpy as jnp
from jax.experimental import pallas as pl
from jax.experimental.pallas import tpu as pltpu

_VQ = 8
_K = 1024
_D = 256
_N = 8192
_H = 256
_NH = 4
_BT = _H * _NH


def _vq_body(gate_ref, x_ref, cb_ref, quant_ref, idx_ref, sse_ref, km_ref,
             e2_ref, ehi_ref, emid_ref, elo_ref):
    @pl.when(pl.program_id(0) == 0)
    def _init_scratch():
        for i in range(_VQ):
            e = cb_ref[i]
            e2_ref[i, :] = jnp.sum(e * e, axis=1)
            hi = e.astype(jnp.bfloat16)
            rm = e - hi.astype(jnp.float32)
            mid = rm.astype(jnp.bfloat16)
            lo = (rm - mid.astype(jnp.float32)).astype(jnp.bfloat16)
            ehi_ref[i] = hi
            emid_ref[i] = mid
            elo_ref[i] = lo

    gate = gate_ref[0]
    lane_iota = jax.lax.broadcasted_iota(jnp.int32, (_H, _K), 1).astype(jnp.float32)
    sse = jnp.zeros((1, 1), jnp.float32)
    r = [x_ref[pl.ds(h * _H, _H), :] * gate for h in range(_NH)]
    quant = [None] * _NH
    for i in range(_VQ):
        e = cb_ref[i]
        e2 = e2_ref[i, :]
        for h in range(_NH):
            rh = r[h]
            km_ref[i, pl.ds(h * _H, _H), :] = rh
            r2 = jnp.sum(rh * rh, axis=1, keepdims=True)
            d = (r2 - jnp.dot(rh + rh, e.T, preferred_element_type=jnp.float32)
                 + e2[None, :])
            dmin = jnp.min(d, axis=1, keepdims=True)
            idx = jnp.min(jnp.where(d == dmin, lane_iota, jnp.float32(_K)), axis=1)
            idx_ref[i, pl.ds(h * _H, _H)] = idx.astype(jnp.int32)
            oh = (lane_iota == idx[:, None]).astype(jnp.float32).astype(jnp.bfloat16)
            q = ((jnp.dot(oh, ehi_ref[i], preferred_element_type=jnp.float32)
                  + jnp.dot(oh, emid_ref[i], preferred_element_type=jnp.float32))
                 + jnp.dot(oh, elo_ref[i], preferred_element_type=jnp.float32))
            diff = q - rh
            sse += jnp.sum(diff * diff).reshape(1, 1)
            q_st = rh + diff
            quant[h] = q_st if i == 0 else quant[h] + q_st
            r[h] = rh - q_st
    for h in range(_NH):
        quant_ref[pl.ds(h * _H, _H), :] = quant[h]

    @pl.when(pl.program_id(0) == 0)
    def _init():
        sse_ref[...] = sse

    @pl.when(pl.program_id(0) != 0)
    def _acc():
        sse_ref[...] += sse


def kernel(x, skip_vq, codebooks):
    gate = (1 - jnp.asarray(skip_vq)).astype(x.dtype).reshape(1)
    xf = x.reshape(_N, _D)
    grid = (_N // _BT,)
    quant, idx, sse, km = pl.pallas_call(
        _vq_body,
        grid=grid,
        in_specs=[
            pl.BlockSpec(memory_space=pltpu.SMEM),
            pl.BlockSpec((_BT, _D), lambda t: (t, 0)),
            pl.BlockSpec((_VQ, _K, _D), lambda t: (0, 0, 0)),
        ],
        out_specs=[
            pl.BlockSpec((_BT, _D), lambda t: (t, 0)),
            pl.BlockSpec((_VQ, _BT), lambda t: (0, t)),
            pl.BlockSpec((1, 1), lambda t: (0, 0)),
            pl.BlockSpec((_VQ, _BT, _D), lambda t: (0, t, 0)),
        ],
        out_shape=[
            jax.ShapeDtypeStruct((_N, _D), jnp.float32),
            jax.ShapeDtypeStruct((_VQ, _N), jnp.int32),
            jax.ShapeDtypeStruct((1, 1), jnp.float32),
            jax.ShapeDtypeStruct((_VQ, _N, _D), jnp.float32),
        ],
        scratch_shapes=[pltpu.VMEM((_VQ, _K), jnp.float32),
                        pltpu.VMEM((_VQ, _K, _D), jnp.bfloat16),
                        pltpu.VMEM((_VQ, _K, _D), jnp.bfloat16),
                        pltpu.VMEM((_VQ, _K, _D), jnp.bfloat16)],
    )(gate, xf, codebooks)
    quantized = quant.reshape(x.shape)
    indices = idx.reshape(_VQ, x.shape[0], x.shape[1])
    vq_loss = (sse / (_N * _D)).reshape(())
    kmeans_inputs = km.reshape(_VQ, x.shape[0], x.shape[1], _D)
    return (quantized, indices, vq_loss, kmeans_inputs)


# speedup vs baseline: 1.7410x; 1.0013x over previous
import jax
import jax.numpy as jnp
from jax.experimental import pallas as pl
from jax.experimental.pallas import tpu as pltpu

_VQ = 8
_K = 1024
_D = 256
_N = 8192
_H = 256
_NH = 4
_BT = _H * _NH


def _vq_body(gate_ref, x_ref, cb_ref, quant_ref, idx_ref, sse_ref, km_ref,
             e2_ref, esplit_ref):
    @pl.when(pl.program_id(0) == 0)
    def _init_scratch():
        for i in range(_VQ):
            e = cb_ref[i]
            e2_ref[i, :] = jnp.sum(e * e, axis=1)
            hi = e.astype(jnp.bfloat16)
            rm = e - hi.astype(jnp.float32)
            mid = rm.astype(jnp.bfloat16)
            lo = (rm - mid.astype(jnp.float32)).astype(jnp.bfloat16)
            esplit_ref[i, :, 0:_D] = hi
            esplit_ref[i, :, _D:2 * _D] = mid
            esplit_ref[i, :, 2 * _D:3 * _D] = lo

    gate = gate_ref[0]
    lane_iota = jax.lax.broadcasted_iota(jnp.int32, (_H, _K), 1).astype(jnp.float32)
    sse = jnp.zeros((1, 1), jnp.float32)
    r = [x_ref[pl.ds(h * _H, _H), :] * gate for h in range(_NH)]
    quant = [None] * _NH
    for i in range(_VQ):
        e = cb_ref[i]
        e2 = e2_ref[i, :]
        for h in range(_NH):
            rh = r[h]
            km_ref[i, pl.ds(h * _H, _H), :] = rh
            r2 = jnp.sum(rh * rh, axis=1, keepdims=True)
            d = (r2 - jnp.dot(rh + rh, e.T, preferred_element_type=jnp.float32)
                 + e2[None, :])
            dmin = jnp.min(d, axis=1, keepdims=True)
            idx = jnp.min(jnp.where(d == dmin, lane_iota, jnp.float32(_K)), axis=1)
            idx_ref[i, pl.ds(h * _H, _H)] = idx.astype(jnp.int32)
            oh = (lane_iota == idx[:, None]).astype(jnp.float32).astype(jnp.bfloat16)
            q3 = jnp.dot(oh, esplit_ref[i], preferred_element_type=jnp.float32)
            q = ((q3[:, 0:_D] + q3[:, _D:2 * _D]) + q3[:, 2 * _D:3 * _D])
            diff = q - rh
            sse += jnp.sum(diff * diff).reshape(1, 1)
            q_st = rh + diff
            quant[h] = q_st if i == 0 else quant[h] + q_st
            r[h] = rh - q_st
    for h in range(_NH):
        quant_ref[pl.ds(h * _H, _H), :] = quant[h]

    @pl.when(pl.program_id(0) == 0)
    def _init():
        sse_ref[...] = sse

    @pl.when(pl.program_id(0) != 0)
    def _acc():
        sse_ref[...] += sse


def kernel(x, skip_vq, codebooks):
    gate = (1 - jnp.asarray(skip_vq)).astype(x.dtype).reshape(1)
    xf = x.reshape(_N, _D)
    grid = (_N // _BT,)
    quant, idx, sse, km = pl.pallas_call(
        _vq_body,
        grid=grid,
        in_specs=[
            pl.BlockSpec(memory_space=pltpu.SMEM),
            pl.BlockSpec((_BT, _D), lambda t: (t, 0)),
            pl.BlockSpec((_VQ, _K, _D), lambda t: (0, 0, 0)),
        ],
        out_specs=[
            pl.BlockSpec((_BT, _D), lambda t: (t, 0)),
            pl.BlockSpec((_VQ, _BT), lambda t: (0, t)),
            pl.BlockSpec((1, 1), lambda t: (0, 0)),
            pl.BlockSpec((_VQ, _BT, _D), lambda t: (0, t, 0)),
        ],
        out_shape=[
            jax.ShapeDtypeStruct((_N, _D), jnp.float32),
            jax.ShapeDtypeStruct((_VQ, _N), jnp.int32),
            jax.ShapeDtypeStruct((1, 1), jnp.float32),
            jax.ShapeDtypeStruct((_VQ, _N, _D), jnp.float32),
        ],
        scratch_shapes=[pltpu.VMEM((_VQ, _K), jnp.float32),
                        pltpu.VMEM((_VQ, _K, 3 * _D), jnp.bfloat16)],
    )(gate, xf, codebooks)
    quantized = quant.reshape(x.shape)
    indices = idx.reshape(_VQ, x.shape[0], x.shape[1])
    vq_loss = (sse / (_N * _D)).reshape(())
    kmeans_inputs = km.reshape(_VQ, x.shape[0], x.shape[1], _D)
    return (quantized, indices, vq_loss, kmeans_inputs)


# speedup vs baseline: 1.7473x; 1.0036x over previous
import jax
import jax.numpy as jnp
from jax.experimental import pallas as pl
from jax.experimental.pallas import tpu as pltpu

_VQ = 8
_K = 1024
_D = 256
_N = 8192
_H = 256
_NH = 4
_BT = _H * _NH


def _vq_body(gate_ref, x_ref, cb_ref, quant_ref, idx_ref, sse_ref, km_ref,
             e2_ref, esplit_ref):
    @pl.when(pl.program_id(0) == 0)
    def _init_scratch():
        for i in range(_VQ):
            e = cb_ref[i]
            e2_ref[i, :] = jnp.sum(e * e, axis=1)
            hi = e.astype(jnp.bfloat16)
            rm = e - hi.astype(jnp.float32)
            mid = rm.astype(jnp.bfloat16)
            lo = (rm - mid.astype(jnp.float32)).astype(jnp.bfloat16)
            esplit_ref[i, :, 0:_D] = hi
            esplit_ref[i, :, _D:2 * _D] = mid
            esplit_ref[i, :, 2 * _D:3 * _D] = lo

    gate = gate_ref[0]
    lane_iota = jax.lax.broadcasted_iota(jnp.int32, (_H, _K), 1).astype(jnp.float32)
    sse_mat = None
    r = [x_ref[pl.ds(h * _H, _H), :] * gate for h in range(_NH)]
    quant = [None] * _NH
    for i in range(_VQ):
        e = cb_ref[i]
        e2 = e2_ref[i, :]
        for h in range(_NH):
            rh = r[h]
            km_ref[i, pl.ds(h * _H, _H), :] = rh
            r2 = jnp.sum(rh * rh, axis=1, keepdims=True)
            d = (r2 - jnp.dot(rh + rh, e.T, preferred_element_type=jnp.float32)
                 + e2[None, :])
            dmin = jnp.min(d, axis=1, keepdims=True)
            idx = jnp.min(jnp.where(d == dmin, lane_iota, jnp.float32(_K)), axis=1)
            idx_ref[i, pl.ds(h * _H, _H)] = idx.astype(jnp.int32)
            oh = (lane_iota == idx[:, None]).astype(jnp.float32).astype(jnp.bfloat16)
            q3 = jnp.dot(oh, esplit_ref[i], preferred_element_type=jnp.float32)
            q = ((q3[:, 0:_D] + q3[:, _D:2 * _D]) + q3[:, 2 * _D:3 * _D])
            diff = q - rh
            sq = diff * diff
            sse_mat = sq if sse_mat is None else sse_mat + sq
            q_st = rh + diff
            quant[h] = q_st if i == 0 else quant[h] + q_st
            r[h] = rh - q_st
    for h in range(_NH):
        quant_ref[pl.ds(h * _H, _H), :] = quant[h]
    sse = jnp.sum(sse_mat).reshape(1, 1)

    @pl.when(pl.program_id(0) == 0)
    def _init():
        sse_ref[...] = sse

    @pl.when(pl.program_id(0) != 0)
    def _acc():
        sse_ref[...] += sse


def kernel(x, skip_vq, codebooks):
    gate = (1 - jnp.asarray(skip_vq)).astype(x.dtype).reshape(1)
    xf = x.reshape(_N, _D)
    grid = (_N // _BT,)
    quant, idx, sse, km = pl.pallas_call(
        _vq_body,
        grid=grid,
        in_specs=[
            pl.BlockSpec(memory_space=pltpu.SMEM),
            pl.BlockSpec((_BT, _D), lambda t: (t, 0)),
            pl.BlockSpec((_VQ, _K, _D), lambda t: (0, 0, 0)),
        ],
        out_specs=[
            pl.BlockSpec((_BT, _D), lambda t: (t, 0)),
            pl.BlockSpec((_VQ, _BT), lambda t: (0, t)),
            pl.BlockSpec((1, 1), lambda t: (0, 0)),
            pl.BlockSpec((_VQ, _BT, _D), lambda t: (0, t, 0)),
        ],
        out_shape=[
            jax.ShapeDtypeStruct((_N, _D), jnp.float32),
            jax.ShapeDtypeStruct((_VQ, _N), jnp.int32),
            jax.ShapeDtypeStruct((1, 1), jnp.float32),
            jax.ShapeDtypeStruct((_VQ, _N, _D), jnp.float32),
        ],
        scratch_shapes=[pltpu.VMEM((_VQ, _K), jnp.float32),
                        pltpu.VMEM((_VQ, _K, 3 * _D), jnp.bfloat16)],
    )(gate, xf, codebooks)
    quantized = quant.reshape(x.shape)
    indices = idx.reshape(_VQ, x.shape[0], x.shape[1])
    vq_loss = (sse / (_N * _D)).reshape(())
    kmeans_inputs = km.reshape(_VQ, x.shape[0], x.shape[1], _D)
    return (quantized, indices, vq_loss, kmeans_inputs)
